# speedup vs baseline: 1.8984x; 1.8984x over previous
import functools

import jax
import jax.numpy as jnp
from jax import lax
from jax.experimental import pallas as pl
from jax.experimental.pallas import tpu as pltpu
from jax.experimental.pallas import tpu_sc as plsc

VOCAB = 8192
D = 8192
NB = 8192
NW = 32
CK = 4
RPW = NB // NW
G = RPW // CK


def _make_gather():
    mesh = plsc.VectorSubcoreMesh(core_axis_name="c", subcore_axis_name="s")
    nc = 2

    @functools.partial(
        pl.kernel,
        mesh=mesh,
        out_type=jax.ShapeDtypeStruct((NB, D), jnp.float32),
        scratch_types=[
            pltpu.VMEM((G, CK), jnp.int32),
            pltpu.VMEM((CK, D), jnp.float32),
            pltpu.VMEM((CK, D), jnp.float32),
            pltpu.SemaphoreType.DMA,
            pltpu.SemaphoreType.DMA,
            pltpu.SemaphoreType.DMA,
            pltpu.SemaphoreType.DMA,
        ],
    )
    def k(table_hbm, idx_hbm, out_hbm, idx_v, buf0, buf1, gsem0, gsem1,
          osem0, osem1):
        wid = lax.axis_index("s") * nc + lax.axis_index("c")
        base = wid * RPW
        pltpu.sync_copy(idx_hbm.at[wid], idx_v)

        bufs = (buf0, buf1)
        gsems = (gsem0, gsem1)
        osems = (osem0, osem1)

        def gather_copy(g, b):
            return pltpu.make_async_copy(
                table_hbm.at[idx_v.at[g]], bufs[b], gsems[b])

        def out_copy(g, b):
            return pltpu.make_async_copy(
                bufs[b], out_hbm.at[pl.ds(base + g * CK, CK)], osems[b])

        gather_copy(0, 0).start()
        gather_copy(1, 1).start()

        def step(s, carry):
            for b in range(2):
                g = 2 * s + b
                gather_copy(g, b).wait()
                out_copy(g, b).start()
            for b in range(2):
                g = 2 * s + b
                out_copy(g, b).wait()
                gather_copy(g + 2, b).start()
            return carry

        lax.fori_loop(0, G // 2 - 1, step, 0, unroll=False)

        for b in range(2):
            g = G - 2 + b
            gather_copy(g, b).wait()
            out_copy(g, b).start()
        for b in range(2):
            g = G - 2 + b
            out_copy(g, b).wait()

    return k


_gather = _make_gather()


def kernel(x, table):
    idx = x.reshape(NW, G, CK).astype(jnp.int32)
    out = _gather(table, idx)
    return out.reshape(x.shape[0], x.shape[1], D)


# speedup vs baseline: 1.9333x; 1.0184x over previous
import functools

import jax
import jax.numpy as jnp
from jax import lax
from jax.experimental import pallas as pl
from jax.experimental.pallas import tpu as pltpu
from jax.experimental.pallas import tpu_sc as plsc

VOCAB = 8192
D = 8192
NB = 8192
NW = 32
CK = 4
NBUF = 3
RPW = NB // NW
G = RPW // CK


def _make_gather():
    mesh = plsc.VectorSubcoreMesh(core_axis_name="c", subcore_axis_name="s")
    nc = 2

    @functools.partial(
        pl.kernel,
        mesh=mesh,
        out_type=jax.ShapeDtypeStruct((NB, D), jnp.float32),
        scratch_types=[
            pltpu.VMEM((G, CK), jnp.int32),
        ] + [pltpu.VMEM((CK, D), jnp.float32)] * NBUF
          + [pltpu.SemaphoreType.DMA] * (2 * NBUF),
    )
    def k(table_hbm, idx_hbm, out_hbm, idx_v, *bufs_sems):
        bufs = bufs_sems[:NBUF]
        gsems = bufs_sems[NBUF:2 * NBUF]
        osems = bufs_sems[2 * NBUF:]
        wid = lax.axis_index("s") * nc + lax.axis_index("c")
        base = wid * RPW
        pltpu.sync_copy(idx_hbm.at[wid], idx_v)

        def gather_copy(g, b):
            return pltpu.make_async_copy(
                table_hbm.at[idx_v.at[g]], bufs[b], gsems[b])

        def out_copy(g, b):
            return pltpu.make_async_copy(
                bufs[b], out_hbm.at[pl.ds(base + g * CK, CK)], osems[b])

        for b in range(NBUF):
            gather_copy(b, b).start()

        def step(s, carry):
            for b in range(NBUF):
                g = NBUF * s + b
                gather_copy(g, b).wait()
                out_copy(g, b).start()
            for b in range(NBUF):
                g = NBUF * s + b
                out_copy(g, b).wait()
                gather_copy(g + NBUF, b).start()
            return carry

        steps = G // NBUF - 1
        lax.fori_loop(0, steps, step, 0, unroll=False)

        tail0 = steps * NBUF
        for g in range(tail0, G):
            b = g % NBUF
            if g >= tail0 + NBUF:
                out_copy(g - NBUF, b).wait()
                gather_copy(g, b).start()
            gather_copy(g, b).wait()
            out_copy(g, b).start()
        for g in range(G - NBUF, G):
            out_copy(g, g % NBUF).wait()

    return k


_gather = _make_gather()


def kernel(x, table):
    idx = x.reshape(NW, G, CK).astype(jnp.int32)
    out = _gather(table, idx)
    return out.reshape(x.shape[0], x.shape[1], D)
